# speedup vs baseline: 1.0690x; 1.0690x over previous
import functools

import jax
import jax.numpy as jnp
from jax import lax
from jax.experimental import pallas as pl
from jax.experimental.pallas import tpu as pltpu
from jax.experimental.pallas import tpu_sc as plsc

B = 4
S = 2048
D = 2048
FLAT = B * S
NC = 2
NS = 16
NW = NC * NS
ROWS_PER_W = FLAT // NW
C = 8
NBUF = 2
NCHUNKS = ROWS_PER_W // C
LANES = 16


def _body(x_hbm, tok_hbm, pos_hbm, out_hbm,
          idx_v, tok_v, pos_v, out_v,
          sem_tok, sem_pos, sem_out):
    wid = lax.axis_index("s") * NC + lax.axis_index("c")
    base = wid * ROWS_PER_W
    pos_base = lax.rem(base, S)

    pltpu.sync_copy(x_hbm.at[pl.ds(base, ROWS_PER_W)], idx_v)

    def issue_in(c, b):
        pltpu.async_copy(
            tok_hbm.at[idx_v.at[pl.ds(c * C, C)]], tok_v.at[b], sem_tok[b])
        pltpu.async_copy(
            pos_hbm.at[pl.ds(pos_base + c * C, C)], pos_v.at[b], sem_pos[b])

    def wait_in(c, b):
        pltpu.make_async_copy(
            tok_hbm.at[idx_v.at[pl.ds(c * C, C)]], tok_v.at[b], sem_tok[b]).wait()
        pltpu.make_async_copy(
            pos_hbm.at[pl.ds(pos_base + c * C, C)], pos_v.at[b], sem_pos[b]).wait()

    def issue_out(c, b):
        pltpu.async_copy(
            out_v.at[b], out_hbm.at[pl.ds(base + c * C, C)], sem_out[b])

    def wait_out(c, b):
        pltpu.make_async_copy(
            out_v.at[b], out_hbm.at[pl.ds(base + c * C, C)], sem_out[b]).wait()

    def add_chunk(b):
        for r in range(C):
            def vbody(j, _, r=r):
                o = j * (4 * LANES)
                for u in range(4):
                    s0 = o + u * LANES
                    out_v[b, r, pl.ds(s0, LANES)] = (
                        tok_v[b, r, pl.ds(s0, LANES)]
                        + pos_v[b, r, pl.ds(s0, LANES)])
                return 0
            lax.fori_loop(0, D // (4 * LANES), vbody, 0)

    for b in range(NBUF):
        issue_in(b, b)

    def outer(g, _):
        for b in range(NBUF):
            c = g * NBUF + b
            wait_in(c, b)

            @pl.when(c >= NBUF)
            def _():
                wait_out(c - NBUF, b)

            add_chunk(b)
            issue_out(c, b)

            @pl.when(c + NBUF < NCHUNKS)
            def _():
                issue_in(c + NBUF, b)
        return 0

    lax.fori_loop(0, NCHUNKS // NBUF, outer, 0)

    for b in range(NBUF):
        wait_out(NCHUNKS - NBUF + b, b)


def _run(xf, token_emb, pos_emb):
    mesh = plsc.VectorSubcoreMesh(core_axis_name="c", subcore_axis_name="s")
    kern = functools.partial(
        pl.kernel,
        mesh=mesh,
        out_type=jax.ShapeDtypeStruct((FLAT, D), jnp.float32),
        scratch_types=[
            pltpu.VMEM((ROWS_PER_W,), jnp.int32),
            pltpu.VMEM((NBUF, C, D), jnp.float32),
            pltpu.VMEM((NBUF, C, D), jnp.float32),
            pltpu.VMEM((NBUF, C, D), jnp.float32),
            [pltpu.SemaphoreType.DMA] * NBUF,
            [pltpu.SemaphoreType.DMA] * NBUF,
            [pltpu.SemaphoreType.DMA] * NBUF,
        ],
    )(_body)
    return kern(xf, token_emb, pos_emb)


def kernel(x, token_emb, pos_emb):
    xf = x.reshape(FLAT).astype(jnp.int32)
    out = _run(xf, token_emb, pos_emb)
    return out.reshape(B, S, D)
